# Initial kernel scaffold; baseline (speedup 1.0000x reference)
import jax
import jax.numpy as jnp
from jax.experimental import pallas as pl


def kernel(w0, b0, w1a, b1a, w1b, b1b, w2a, b2a, w2b, b2b, wr, br, ctx, res):
    raise NotImplementedError("write your pallas kernel here")


# speedup vs baseline: 1.4909x; 1.4909x over previous
import jax
import jax.numpy as jnp
import numpy as np
from jax import lax
from jax.experimental import pallas as pl
from jax.experimental.pallas import tpu as pltpu

_SLOPE = 0.01
_TILE = 256
_SLOT = 128


def _rup(x, m):
    return ((x + m - 1) // m) * m


def _w9(w, slot):
    cin = w.shape[2]
    wp = jnp.pad(w, ((0, 0), (0, 0), (0, slot - cin), (0, 0)))
    return wp.reshape(9 * slot, w.shape[3])


def kernel(w0, b0, w1a, b1a, w1b, b1b, w2a, b2a, w2b, b2b, wr, br, ctx, res):
    B, Cc, H, W = ctx.shape
    Cr = res.shape[1]
    Cin0 = Cc + Cr
    C = w0.shape[-1]
    CR = wr.shape[-1]
    Hp, Wp = H + 2, W + 2
    Npi = Hp * Wp
    NT = -(-Npi // _TILE)
    NPAD = NT * _TILE
    G = _rup(NPAD - Npi + Wp + 2, 8)
    ROWS = Npi + 2 * G
    KW = 9 * _SLOT
    f32 = jnp.float32
    bf16 = jnp.bfloat16

    x = jnp.concatenate([jnp.transpose(ctx, (0, 2, 3, 1)),
                         jnp.transpose(res, (0, 2, 3, 1))], axis=-1)
    xp = jnp.pad(x, ((0, 0), (1, 1), (1, 1), (0, 0))).reshape(B, Npi, Cin0)
    xg = jnp.pad(xp, ((0, 0), (G, G), (0, 0)))

    ys = np.arange(Hp)[:, None]
    xs = np.arange(Wp)[None, :]
    inter = ((ys >= 1) & (ys <= H) & (xs >= 1) & (xs <= W)).astype(np.float32)
    mask_np = np.zeros((NPAD, 1), np.float32)
    mask_np[:Npi, 0] = inter.reshape(-1)
    mask = jnp.asarray(mask_np)

    w0m = _w9(w0, _SLOT).astype(bf16)
    wcm = jnp.stack([_w9(w, _SLOT) for w in (w1a, w1b, w2a, w2b)]).astype(bf16)
    wrm = _w9(wr, _SLOT).astype(bf16)
    bm = jnp.stack([b.reshape(1, C).astype(f32)
                    for b in (b0, b1a, b1b, b2a, b2b)])
    brm = br.reshape(1, CR).astype(f32)

    def body(mask_ref, x_ref, w0_ref, wc_ref, wr_ref, bm_ref, br_ref,
             feat_ref, rec_ref, act_ref, pb_ref, slab_ref):

        def lrelu(v):
            return jnp.where(v >= 0, v, _SLOPE * v)

        def conv_pass(src_ref, cin, w, bias, store):
            def tile(t, carry):
                r0 = pl.multiple_of(t * _TILE, _TILE)
                for j in range(9):
                    d = (j // 3 - 1) * Wp + (j % 3 - 1)
                    slab_ref[:, j * _SLOT:j * _SLOT + cin] = (
                        src_ref[pl.ds(G + d + r0, _TILE), 0:cin].astype(bf16))
                y = jnp.dot(slab_ref[...], w,
                            preferred_element_type=f32) + bias
                store(r0, y)
                return carry
            lax.fori_loop(0, NT, tile, 0)

        slab_ref[...] = jnp.zeros_like(slab_ref)
        tail = ROWS - (G + NPAD)
        for ref in (feat_ref, act_ref, pb_ref):
            ref[0:G, :] = jnp.zeros((G, C), f32)
            ref[G + NPAD:ROWS, :] = jnp.zeros((tail, C), f32)

        def st_head(r0, y):
            m = mask_ref[pl.ds(r0, _TILE), :]
            y = y * m
            feat_ref[pl.ds(G + r0, _TILE), :] = y
            act_ref[pl.ds(G + r0, _TILE), :] = lrelu(y)

        def st_mid(r0, y):
            m = mask_ref[pl.ds(r0, _TILE), :]
            pb_ref[pl.ds(G + r0, _TILE), :] = lrelu(y) * m

        def st_res(r0, y):
            m = mask_ref[pl.ds(r0, _TILE), :]
            y = y * m + feat_ref[pl.ds(G + r0, _TILE), :]
            feat_ref[pl.ds(G + r0, _TILE), :] = y
            act_ref[pl.ds(G + r0, _TILE), :] = lrelu(y)

        def st_rec(r0, y):
            rec_ref[pl.ds(r0, _TILE), :] = y

        conv_pass(x_ref, Cin0, w0_ref[...], bm_ref[0], st_head)
        conv_pass(act_ref, C, wc_ref[0], bm_ref[1], st_mid)
        conv_pass(pb_ref, C, wc_ref[1], bm_ref[2], st_res)
        conv_pass(act_ref, C, wc_ref[2], bm_ref[3], st_mid)
        conv_pass(pb_ref, C, wc_ref[3], bm_ref[4], st_res)
        conv_pass(feat_ref, C, wr_ref[...], br_ref[...], st_rec)

    feat, rec = pl.pallas_call(
        body,
        out_shape=(jax.ShapeDtypeStruct((B, ROWS, C), f32),
                   jax.ShapeDtypeStruct((B, NPAD, CR), f32)),
        grid=(B,),
        in_specs=[
            pl.BlockSpec((NPAD, 1), lambda b: (0, 0)),
            pl.BlockSpec((None, ROWS, Cin0), lambda b: (b, 0, 0)),
            pl.BlockSpec((KW, C), lambda b: (0, 0)),
            pl.BlockSpec((4, KW, C), lambda b: (0, 0, 0)),
            pl.BlockSpec((KW, CR), lambda b: (0, 0)),
            pl.BlockSpec((5, 1, C), lambda b: (0, 0, 0)),
            pl.BlockSpec((1, CR), lambda b: (0, 0)),
        ],
        out_specs=(pl.BlockSpec((None, ROWS, C), lambda b: (b, 0, 0)),
                   pl.BlockSpec((None, NPAD, CR), lambda b: (b, 0, 0))),
        scratch_shapes=[pltpu.VMEM((ROWS, C), f32),
                        pltpu.VMEM((ROWS, C), f32),
                        pltpu.VMEM((_TILE, KW), bf16)],
        compiler_params=pltpu.CompilerParams(
            dimension_semantics=("parallel",),
            vmem_limit_bytes=100 << 20),
    )(mask, xg, w0m, wcm, wrm, bm, brm)

    ft = feat[:, G:G + Npi, :].reshape(B, Hp, Wp, C)[:, 1:H + 1, 1:W + 1, :]
    rc = rec[:, :Npi, :].reshape(B, Hp, Wp, CR)[:, 1:H + 1, 1:W + 1, :]
    ft = jnp.transpose(ft, (0, 3, 1, 2)).astype(ctx.dtype)
    rc = jnp.transpose(rc, (0, 3, 1, 2)).astype(ctx.dtype)
    return ft, rc


# speedup vs baseline: 1.8116x; 1.2151x over previous
import jax
import jax.numpy as jnp
import numpy as np
from jax import lax
from jax.experimental import pallas as pl
from jax.experimental.pallas import tpu as pltpu

_SLOPE = 0.01
_TILE = 256
_SLOT = 128
_S = 112


def _rup(x, m):
    return ((x + m - 1) // m) * m


def kernel(w0, b0, w1a, b1a, w1b, b1b, w2a, b2a, w2b, b2b, wr, br, ctx, res):
    B, Cc, H, W = ctx.shape
    Cr = res.shape[1]
    Cin0 = Cc + Cr
    C = w0.shape[-1]
    CR = wr.shape[-1]
    Hp = H + 2
    Npi = Hp * _S
    NT = 2 * (-(-Npi // (2 * _TILE)))
    NPAD = NT * _TILE
    G = _rup(max(NPAD - Npi, 0) + _S + 16, 16)
    ROWS = Npi + 2 * G
    MS = _TILE + 32
    KW = 3 * _SLOT
    NW = 3 * C
    f32 = jnp.float32
    bf16 = jnp.bfloat16

    x = jnp.concatenate([jnp.transpose(ctx, (0, 2, 3, 1)),
                         jnp.transpose(res, (0, 2, 3, 1))], axis=-1).astype(bf16)
    xp = jnp.pad(x, ((0, 0), (1, 1), (1, _S - W - 1), (0, 0)))
    xg = jnp.pad(xp.reshape(B, Npi, Cin0), ((0, 0), (G, G), (0, 0)))

    hh = np.arange(Hp)[:, None]
    ww = np.arange(_S)[None, :]
    inter = ((hh >= 1) & (hh <= H) & (ww >= 1) & (ww <= W)).astype(np.float32)
    mask_np = np.zeros((NPAD, 1), np.float32)
    mask_np[:Npi, 0] = inter.reshape(-1)
    mask = jnp.asarray(mask_np)

    def _wcat(w, cout_pad):
        cin, cout = w.shape[2], w.shape[3]
        wp = jnp.pad(w, ((0, 0), (0, 0), (0, _SLOT - cin), (0, cout_pad - cout)))
        return jnp.transpose(wp, (0, 2, 1, 3)).reshape(3 * _SLOT, 3 * cout_pad)

    w0m = _wcat(w0, C).astype(bf16)
    wcm = jnp.stack([_wcat(w, C) for w in (w1a, w1b, w2a, w2b)]).astype(bf16)
    wrm = _wcat(wr, C).astype(bf16)
    bm = jnp.stack([b.reshape(1, C).astype(f32)
                    for b in (b0, b1a, b1b, b2a, b2b)])
    brm = jnp.pad(br, (0, C - CR)).reshape(1, C).astype(f32)

    def body(mask_ref, x_ref, w0_ref, wc_ref, wr_ref, bm_ref, br_ref,
             feat_ref, rec_ref, act_ref, pb_ref, slab0, slab1):

        def lrelu(v):
            return jnp.where(v >= 0, v, _SLOPE * v)

        def conv_tile(src_ref, cin, w, bias, store, r0, slab_ref):
            for ky in range(3):
                base = G + (ky - 1) * _S - 16 + r0
                slab_ref[:, ky * _SLOT:ky * _SLOT + cin] = (
                    src_ref[pl.ds(base, MS), 0:cin])
            p = jnp.dot(slab_ref[...], w, preferred_element_type=f32)
            y = (p[15:15 + _TILE, 0:C] + p[16:16 + _TILE, C:2 * C]
                 + p[17:17 + _TILE, 2 * C:3 * C]) + bias
            store(r0, y)

        def conv_pass(src_ref, cin, w, bias, store):
            def two(i, carry):
                r0 = pl.multiple_of(i * (2 * _TILE), 2 * _TILE)
                conv_tile(src_ref, cin, w, bias, store, r0, slab0)
                conv_tile(src_ref, cin, w, bias, store, r0 + _TILE, slab1)
                return carry
            lax.fori_loop(0, NT // 2, two, 0)

        slab0[...] = jnp.zeros_like(slab0)
        slab1[...] = jnp.zeros_like(slab1)
        tail = ROWS - (G + NPAD)
        for ref in (feat_ref, act_ref, pb_ref):
            ref[0:G, :] = jnp.zeros((G, C), bf16)
            ref[G + NPAD:ROWS, :] = jnp.zeros((tail, C), bf16)

        def st_head(r0, y):
            m = mask_ref[pl.ds(r0, _TILE), :]
            y = y * m
            feat_ref[pl.ds(G + r0, _TILE), :] = y.astype(bf16)
            act_ref[pl.ds(G + r0, _TILE), :] = lrelu(y).astype(bf16)

        def st_mid(r0, y):
            m = mask_ref[pl.ds(r0, _TILE), :]
            pb_ref[pl.ds(G + r0, _TILE), :] = (lrelu(y) * m).astype(bf16)

        def st_res(r0, y):
            m = mask_ref[pl.ds(r0, _TILE), :]
            y = y * m + feat_ref[pl.ds(G + r0, _TILE), :].astype(f32)
            feat_ref[pl.ds(G + r0, _TILE), :] = y.astype(bf16)
            act_ref[pl.ds(G + r0, _TILE), :] = lrelu(y).astype(bf16)

        def st_rec(r0, y):
            rec_ref[pl.ds(r0, _TILE), :] = y[:, 0:CR].astype(bf16)

        conv_pass(x_ref, Cin0, w0_ref[...], bm_ref[0], st_head)
        conv_pass(act_ref, C, wc_ref[0], bm_ref[1], st_mid)
        conv_pass(pb_ref, C, wc_ref[1], bm_ref[2], st_res)
        conv_pass(act_ref, C, wc_ref[2], bm_ref[3], st_mid)
        conv_pass(pb_ref, C, wc_ref[3], bm_ref[4], st_res)
        conv_pass(feat_ref, C, wr_ref[...], br_ref[...], st_rec)

    feat, rec = pl.pallas_call(
        body,
        out_shape=(jax.ShapeDtypeStruct((B, ROWS, C), bf16),
                   jax.ShapeDtypeStruct((B, NPAD, CR), bf16)),
        grid=(B,),
        in_specs=[
            pl.BlockSpec((NPAD, 1), lambda b: (0, 0)),
            pl.BlockSpec((None, ROWS, Cin0), lambda b: (b, 0, 0)),
            pl.BlockSpec((KW, NW), lambda b: (0, 0)),
            pl.BlockSpec((4, KW, NW), lambda b: (0, 0, 0)),
            pl.BlockSpec((KW, NW), lambda b: (0, 0)),
            pl.BlockSpec((5, 1, C), lambda b: (0, 0, 0)),
            pl.BlockSpec((1, C), lambda b: (0, 0)),
        ],
        out_specs=(pl.BlockSpec((None, ROWS, C), lambda b: (b, 0, 0)),
                   pl.BlockSpec((None, NPAD, CR), lambda b: (b, 0, 0))),
        scratch_shapes=[pltpu.VMEM((ROWS, C), bf16),
                        pltpu.VMEM((ROWS, C), bf16),
                        pltpu.VMEM((MS, KW), bf16),
                        pltpu.VMEM((MS, KW), bf16)],
        compiler_params=pltpu.CompilerParams(
            dimension_semantics=("parallel",),
            vmem_limit_bytes=100 << 20),
    )(mask, xg, w0m, wcm, wrm, bm, brm)

    ft = feat[:, G:G + Npi, :].reshape(B, Hp, _S, C)[:, 1:H + 1, 1:W + 1, :]
    rc = rec[:, :Npi, :].reshape(B, Hp, _S, CR)[:, 1:H + 1, 1:W + 1, :]
    ft = jnp.transpose(ft, (0, 3, 1, 2)).astype(ctx.dtype)
    rc = jnp.transpose(rc, (0, 3, 1, 2)).astype(ctx.dtype)
    return ft, rc


# speedup vs baseline: 2.1166x; 1.1684x over previous
import jax
import jax.numpy as jnp
import numpy as np
from jax import lax
from jax.experimental import pallas as pl
from jax.experimental.pallas import tpu as pltpu

_SLOPE = 0.01
_TILE = 512
_SLOT = 128
_S = 112


def _rup(x, m):
    return ((x + m - 1) // m) * m


def kernel(w0, b0, w1a, b1a, w1b, b1b, w2a, b2a, w2b, b2b, wr, br, ctx, res):
    B, Cc, H, W = ctx.shape
    Cr = res.shape[1]
    Cin0 = Cc + Cr
    C = w0.shape[-1]
    CR = wr.shape[-1]
    Hp = H + 2
    Npi = Hp * _S
    NT = 2 * (-(-Npi // (2 * _TILE)))
    NPAD = NT * _TILE
    G = _rup(max(NPAD - Npi, 0) + _S + 16, 16)
    ROWS = Npi + 2 * G
    MS = _TILE + 32
    KW = 3 * _SLOT
    NW = 3 * C
    f32 = jnp.float32
    bf16 = jnp.bfloat16

    x = jnp.concatenate([jnp.transpose(ctx, (0, 2, 3, 1)),
                         jnp.transpose(res, (0, 2, 3, 1))], axis=-1).astype(bf16)
    xp = jnp.pad(x, ((0, 0), (1, 1), (1, _S - W - 1), (0, 0)))
    xg = jnp.pad(xp.reshape(B, Npi, Cin0), ((0, 0), (G, G), (0, 0)))

    hh = np.arange(Hp)[:, None]
    ww = np.arange(_S)[None, :]
    inter = ((hh >= 1) & (hh <= H) & (ww >= 1) & (ww <= W)).astype(np.float32)
    mask_np = np.zeros((NPAD, 1), np.float32)
    mask_np[:Npi, 0] = inter.reshape(-1)
    mask = jnp.asarray(mask_np)

    def _wcat(w, cout_pad):
        cin, cout = w.shape[2], w.shape[3]
        wp = jnp.pad(w, ((0, 0), (0, 0), (0, _SLOT - cin), (0, cout_pad - cout)))
        return jnp.transpose(wp, (0, 2, 1, 3)).reshape(3 * _SLOT, 3 * cout_pad)

    w0m = _wcat(w0, C).astype(bf16)
    wcm = jnp.stack([_wcat(w, C) for w in (w1a, w1b, w2a, w2b)]).astype(bf16)
    wrm = _wcat(wr, C).astype(bf16)
    bm = jnp.stack([b.reshape(1, C).astype(f32)
                    for b in (b0, b1a, b1b, b2a, b2b)])
    brm = jnp.pad(br, (0, C - CR)).reshape(1, C).astype(f32)

    def body(mask_ref, x_ref, w0_ref, wc_ref, wr_ref, bm_ref, br_ref,
             feat_ref, rec_ref, act_ref, pb_ref, slab0, slab1):

        def lrelu(v):
            return jnp.where(v >= 0, v, _SLOPE * v)

        def conv_tile(src_ref, cin, w, bias, store, r0, slab_ref):
            for ky in range(3):
                base = G + (ky - 1) * _S - 16 + r0
                slab_ref[:, ky * _SLOT:ky * _SLOT + cin] = (
                    src_ref[pl.ds(base, MS), 0:cin])
            p = jnp.dot(slab_ref[...], w, preferred_element_type=f32)
            y = (p[15:15 + _TILE, 0:C] + p[16:16 + _TILE, C:2 * C]
                 + p[17:17 + _TILE, 2 * C:3 * C]) + bias
            store(r0, y)

        def conv_pass(src_ref, cin, w, bias, store):
            def two(i, carry):
                r0 = pl.multiple_of(i * (2 * _TILE), 2 * _TILE)
                conv_tile(src_ref, cin, w, bias, store, r0, slab0)
                conv_tile(src_ref, cin, w, bias, store, r0 + _TILE, slab1)
                return carry
            lax.fori_loop(0, NT // 2, two, 0)

        slab0[...] = jnp.zeros_like(slab0)
        slab1[...] = jnp.zeros_like(slab1)
        tail = ROWS - (G + NPAD)
        for ref in (feat_ref, act_ref, pb_ref):
            ref[0:G, :] = jnp.zeros((G, C), bf16)
            ref[G + NPAD:ROWS, :] = jnp.zeros((tail, C), bf16)

        def st_head(r0, y):
            m = mask_ref[pl.ds(r0, _TILE), :]
            y = y * m
            feat_ref[pl.ds(G + r0, _TILE), :] = y.astype(bf16)
            act_ref[pl.ds(G + r0, _TILE), :] = lrelu(y).astype(bf16)

        def st_mid(r0, y):
            m = mask_ref[pl.ds(r0, _TILE), :]
            pb_ref[pl.ds(G + r0, _TILE), :] = (lrelu(y) * m).astype(bf16)

        def st_res(r0, y):
            m = mask_ref[pl.ds(r0, _TILE), :]
            y = y * m + feat_ref[pl.ds(G + r0, _TILE), :].astype(f32)
            feat_ref[pl.ds(G + r0, _TILE), :] = y.astype(bf16)
            act_ref[pl.ds(G + r0, _TILE), :] = lrelu(y).astype(bf16)

        def st_rec(r0, y):
            rec_ref[pl.ds(r0, _TILE), :] = y[:, 0:CR].astype(bf16)

        conv_pass(x_ref, Cin0, w0_ref[...], bm_ref[0], st_head)
        conv_pass(act_ref, C, wc_ref[0], bm_ref[1], st_mid)
        conv_pass(pb_ref, C, wc_ref[1], bm_ref[2], st_res)
        conv_pass(act_ref, C, wc_ref[2], bm_ref[3], st_mid)
        conv_pass(pb_ref, C, wc_ref[3], bm_ref[4], st_res)
        conv_pass(feat_ref, C, wr_ref[...], br_ref[...], st_rec)

    feat, rec = pl.pallas_call(
        body,
        out_shape=(jax.ShapeDtypeStruct((B, ROWS, C), bf16),
                   jax.ShapeDtypeStruct((B, NPAD, CR), bf16)),
        grid=(B,),
        in_specs=[
            pl.BlockSpec((NPAD, 1), lambda b: (0, 0)),
            pl.BlockSpec((None, ROWS, Cin0), lambda b: (b, 0, 0)),
            pl.BlockSpec((KW, NW), lambda b: (0, 0)),
            pl.BlockSpec((4, KW, NW), lambda b: (0, 0, 0)),
            pl.BlockSpec((KW, NW), lambda b: (0, 0)),
            pl.BlockSpec((5, 1, C), lambda b: (0, 0, 0)),
            pl.BlockSpec((1, C), lambda b: (0, 0)),
        ],
        out_specs=(pl.BlockSpec((None, ROWS, C), lambda b: (b, 0, 0)),
                   pl.BlockSpec((None, NPAD, CR), lambda b: (b, 0, 0))),
        scratch_shapes=[pltpu.VMEM((ROWS, C), bf16),
                        pltpu.VMEM((ROWS, C), bf16),
                        pltpu.VMEM((MS, KW), bf16),
                        pltpu.VMEM((MS, KW), bf16)],
        compiler_params=pltpu.CompilerParams(
            dimension_semantics=("parallel",),
            vmem_limit_bytes=100 << 20),
    )(mask, xg, w0m, wcm, wrm, bm, brm)

    ft = feat[:, G:G + Npi, :].reshape(B, Hp, _S, C)[:, 1:H + 1, 1:W + 1, :]
    rc = rec[:, :Npi, :].reshape(B, Hp, _S, CR)[:, 1:H + 1, 1:W + 1, :]
    ft = jnp.transpose(ft, (0, 3, 1, 2)).astype(ctx.dtype)
    rc = jnp.transpose(rc, (0, 3, 1, 2)).astype(ctx.dtype)
    return ft, rc


# speedup vs baseline: 3.8189x; 1.8042x over previous
import jax
import jax.numpy as jnp
from jax import lax
from jax.experimental import pallas as pl
from jax.experimental.pallas import tpu as pltpu

_SLOPE = 0.01
_SL = 128
_TN = 896
_GL = 128


def kernel(w0, b0, w1a, b1a, w1b, b1b, w2a, b2a, w2b, b2b, wr, br, ctx, res):
    B, Cc, H, W = ctx.shape
    Cr = res.shape[1]
    Cin0 = Cc + Cr
    C = w0.shape[-1]
    CR = wr.shape[-1]
    Hp = H + 2
    Lp = Hp * _SL
    NT = 2 * (-(-Lp // (2 * _TN)))
    NPAD = NT * _TN
    L = _GL + NPAD + _GL
    KW0 = 3 * Cin0
    KWC = 3 * C
    MW = 3 * C
    f32 = jnp.float32
    bf16 = jnp.bfloat16

    x4 = jnp.concatenate([ctx, res], axis=1).astype(bf16)
    xp = jnp.pad(x4, ((0, 0), (0, 0), (1, 1), (1, _SL - W - 1)))
    xg = jnp.pad(xp.reshape(B, Cin0, Lp), ((0, 0), (0, 0), (_GL, L - _GL - Lp)))

    def _wT(w):
        cin, cout = w.shape[2], w.shape[3]
        wp = jnp.pad(w, ((0, 0), (0, 0), (0, 0), (0, C - cout)))
        return jnp.transpose(wp, (1, 3, 0, 2)).reshape(3 * C, 3 * cin)

    w0m = _wT(w0).astype(bf16)
    wcm = jnp.stack([_wT(w) for w in (w1a, w1b, w2a, w2b)]).astype(bf16)
    wrm = _wT(wr).astype(bf16)
    bm = jnp.stack([b.reshape(C, 1).astype(f32)
                    for b in (b0, b1a, b1b, b2a, b2b)])
    brm = jnp.pad(br, (0, C - CR)).reshape(C, 1).astype(f32)

    def body(x_ref, w0_ref, wc_ref, wr_ref, bm_ref, br_ref,
             feat_ref, rec_ref, act_ref, pb_ref):

        def lrelu(v):
            return jnp.where(v >= 0, v, _SLOPE * v)

        def interior(q0):
            l = q0 + lax.broadcasted_iota(jnp.int32, (1, _TN), 1)
            h = l >> 7
            w_ = l & 127
            return (h >= 1) & (h <= H) & (w_ >= 1) & (w_ <= W)

        def rolled_sum(p):
            p0, p1, p2 = p[0:C, :], p[C:2 * C, :], p[2 * C:3 * C, :]
            r0 = jnp.concatenate([p0[:, _TN - 1:], p0[:, :_TN - 1]], axis=1)
            r2 = jnp.concatenate([p2[:, 1:], p2[:, :1]], axis=1)
            return r0 + p1 + r2

        def tap_rhs(src_ref, cin, q0):
            v = src_ref[0:cin, pl.ds(_GL - _SL + q0, _TN + 2 * _SL)]
            return jnp.concatenate(
                [v[:, ky * _SL:ky * _SL + _TN] for ky in range(3)], axis=0)

        def conv_pass(src_ref, cin, w, bias, store):
            def two(i, carry):
                q0 = pl.multiple_of(i * (2 * _TN), 2 * _TN)
                p0 = jnp.dot(w, tap_rhs(src_ref, cin, q0),
                             preferred_element_type=f32)
                store(q0, rolled_sum(p0) + bias)
                p1 = jnp.dot(w, tap_rhs(src_ref, cin, q0 + _TN),
                             preferred_element_type=f32)
                store(q0 + _TN, rolled_sum(p1) + bias)
                return carry
            lax.fori_loop(0, NT // 2, two, 0)

        for ref in (feat_ref, act_ref, pb_ref):
            ref[:, 0:_GL] = jnp.zeros((C, _GL), bf16)
            ref[:, _GL + NPAD:L] = jnp.zeros((C, L - _GL - NPAD), bf16)

        def st_head(q0, y):
            y = jnp.where(interior(q0), y, 0.0)
            feat_ref[:, pl.ds(_GL + q0, _TN)] = y.astype(bf16)
            act_ref[:, pl.ds(_GL + q0, _TN)] = lrelu(y).astype(bf16)

        def st_mid(q0, y):
            y = jnp.where(interior(q0), lrelu(y), 0.0)
            pb_ref[:, pl.ds(_GL + q0, _TN)] = y.astype(bf16)

        def st_res(q0, y):
            y = (jnp.where(interior(q0), y, 0.0)
                 + feat_ref[:, pl.ds(_GL + q0, _TN)].astype(f32))
            feat_ref[:, pl.ds(_GL + q0, _TN)] = y.astype(bf16)
            act_ref[:, pl.ds(_GL + q0, _TN)] = lrelu(y).astype(bf16)

        def st_rec(q0, y):
            rec_ref[0:CR, pl.ds(q0, _TN)] = y[0:CR, :].astype(bf16)

        conv_pass(x_ref, Cin0, w0_ref[...], bm_ref[0], st_head)
        conv_pass(act_ref, C, wc_ref[0], bm_ref[1], st_mid)
        conv_pass(pb_ref, C, wc_ref[1], bm_ref[2], st_res)
        conv_pass(act_ref, C, wc_ref[2], bm_ref[3], st_mid)
        conv_pass(pb_ref, C, wc_ref[3], bm_ref[4], st_res)
        conv_pass(feat_ref, C, wr_ref[...], br_ref[...], st_rec)

    feat, rec = pl.pallas_call(
        body,
        out_shape=(jax.ShapeDtypeStruct((B, C, L), bf16),
                   jax.ShapeDtypeStruct((B, CR, NPAD), bf16)),
        grid=(B,),
        in_specs=[
            pl.BlockSpec((None, Cin0, L), lambda b: (b, 0, 0)),
            pl.BlockSpec((MW, KW0), lambda b: (0, 0)),
            pl.BlockSpec((4, MW, KWC), lambda b: (0, 0, 0)),
            pl.BlockSpec((MW, KWC), lambda b: (0, 0)),
            pl.BlockSpec((5, C, 1), lambda b: (0, 0, 0)),
            pl.BlockSpec((C, 1), lambda b: (0, 0)),
        ],
        out_specs=(pl.BlockSpec((None, C, L), lambda b: (b, 0, 0)),
                   pl.BlockSpec((None, CR, NPAD), lambda b: (b, 0, 0))),
        scratch_shapes=[pltpu.VMEM((C, L), bf16),
                        pltpu.VMEM((C, L), bf16)],
        compiler_params=pltpu.CompilerParams(
            dimension_semantics=("parallel",),
            vmem_limit_bytes=100 << 20),
    )(xg, w0m, wcm, wrm, bm, brm)

    ft = feat[:, :, _GL:_GL + Lp].reshape(B, C, Hp, _SL)[:, :, 1:H + 1, 1:W + 1]
    rc = rec[:, :, :Lp].reshape(B, CR, Hp, _SL)[:, :, 1:H + 1, 1:W + 1]
    return ft.astype(ctx.dtype), rc.astype(ctx.dtype)


# speedup vs baseline: 4.2370x; 1.1095x over previous
import jax
import jax.numpy as jnp
from jax import lax
from jax.experimental import pallas as pl
from jax.experimental.pallas import tpu as pltpu

_SLOPE = 0.01
_SL = 128
_TN = 896
_GL = 128


def kernel(w0, b0, w1a, b1a, w1b, b1b, w2a, b2a, w2b, b2b, wr, br, ctx, res):
    B, Cc, H, W = ctx.shape
    Cr = res.shape[1]
    Cin0 = Cc + Cr
    C = w0.shape[-1]
    CR = wr.shape[-1]
    Hp = H + 2
    Lp = Hp * _SL
    NT = 2 * (-(-Lp // (2 * _TN)))
    NPAD = NT * _TN
    L = _GL + NPAD + _GL
    KW0 = 3 * Cin0
    KWC = 3 * C
    MW = 3 * C
    f32 = jnp.float32
    bf16 = jnp.bfloat16

    ctf = ctx.reshape(B, Cc, H * W)
    ref_ = res.reshape(B, Cr, H * W)

    def _wT(w):
        cin, cout = w.shape[2], w.shape[3]
        wp = jnp.pad(w, ((0, 0), (0, 0), (0, 0), (0, C - cout)))
        return jnp.transpose(wp, (1, 3, 0, 2)).reshape(3 * C, 3 * cin)

    w0m = _wT(w0).astype(bf16)
    wcm = jnp.stack([_wT(w) for w in (w1a, w1b, w2a, w2b)]).astype(bf16)
    wrm = _wT(wr).astype(bf16)
    bm = jnp.stack([b.reshape(C, 1).astype(f32)
                    for b in (b0, b1a, b1b, b2a, b2b)])
    brm = jnp.pad(br, (0, C - CR)).reshape(C, 1).astype(f32)

    def body(ct_ref, rs_ref, w0_ref, wc_ref, wr_ref, bm_ref, br_ref,
             feat_ref, rec_ref, x_ref, act_ref, pb_ref):

        def lrelu(v):
            return jnp.where(v >= 0, v, _SLOPE * v)

        def interior(q0):
            l = q0 + lax.broadcasted_iota(jnp.int32, (1, _TN), 1)
            h = l >> 7
            w_ = l & 127
            return (h >= 1) & (h <= H) & (w_ >= 1) & (w_ <= W)

        def rolled_sum(p):
            p0, p1, p2 = p[0:C, :], p[C:2 * C, :], p[2 * C:3 * C, :]
            r0 = jnp.concatenate([p0[:, _TN - 1:], p0[:, :_TN - 1]], axis=1)
            r2 = jnp.concatenate([p2[:, 1:], p2[:, :1]], axis=1)
            return r0 + p1 + r2

        def tap_rhs(src_ref, cin, q0):
            v = src_ref[0:cin, pl.ds(_GL - _SL + q0, _TN + 2 * _SL)]
            return jnp.concatenate(
                [v[:, ky * _SL:ky * _SL + _TN] for ky in range(3)], axis=0)

        def conv_pass(src_ref, cin, w, bias, store):
            def two(i, carry):
                q0 = pl.multiple_of(i * (2 * _TN), 2 * _TN)
                p0 = jnp.dot(w, tap_rhs(src_ref, cin, q0),
                             preferred_element_type=f32)
                store(q0, rolled_sum(p0) + bias)
                p1 = jnp.dot(w, tap_rhs(src_ref, cin, q0 + _TN),
                             preferred_element_type=f32)
                store(q0 + _TN, rolled_sum(p1) + bias)
                return carry
            lax.fori_loop(0, NT // 2, two, 0)

        for ref in (feat_ref, act_ref, pb_ref):
            ref[:, 0:_GL] = jnp.zeros((C, _GL), bf16)
            ref[:, _GL + NPAD:L] = jnp.zeros((C, L - _GL - NPAD), bf16)

        x_ref[:, 0:_GL + _SL] = jnp.zeros((Cin0, _GL + _SL), bf16)
        x_ref[:, _GL + _SL + H * _SL:L] = (
            jnp.zeros((Cin0, L - _GL - _SL - H * _SL), bf16))
        RB = 4 * W
        DB = 4 * _SL

        def pad4(i, carry):
            so = pl.multiple_of(i * RB, 128)
            do = pl.multiple_of(i * DB, 128)
            vc = ct_ref[:, pl.ds(so, RB)].astype(bf16)
            vr = rs_ref[:, pl.ds(so, RB)].astype(bf16)

            def blk(v):
                return jnp.concatenate(
                    [jnp.pad(v[:, j * W:(j + 1) * W], ((0, 0), (1, _SL - W - 1)))
                     for j in range(4)], axis=1)

            x_ref[0:Cc, pl.ds(_GL + _SL + do, DB)] = blk(vc)
            x_ref[Cc:Cin0, pl.ds(_GL + _SL + do, DB)] = blk(vr)
            return carry
        lax.fori_loop(0, H // 4, pad4, 0)

        def st_head(q0, y):
            y = jnp.where(interior(q0), y, 0.0)
            feat_ref[:, pl.ds(_GL + q0, _TN)] = y.astype(bf16)
            act_ref[:, pl.ds(_GL + q0, _TN)] = lrelu(y).astype(bf16)

        def st_mid(q0, y):
            y = jnp.where(interior(q0), lrelu(y), 0.0)
            pb_ref[:, pl.ds(_GL + q0, _TN)] = y.astype(bf16)

        def st_res(q0, y):
            y = (jnp.where(interior(q0), y, 0.0)
                 + feat_ref[:, pl.ds(_GL + q0, _TN)].astype(f32))
            feat_ref[:, pl.ds(_GL + q0, _TN)] = y.astype(bf16)
            act_ref[:, pl.ds(_GL + q0, _TN)] = lrelu(y).astype(bf16)

        def st_rec(q0, y):
            rec_ref[0:CR, pl.ds(q0, _TN)] = y[0:CR, :].astype(bf16)

        conv_pass(x_ref, Cin0, w0_ref[...], bm_ref[0], st_head)
        conv_pass(act_ref, C, wc_ref[0], bm_ref[1], st_mid)
        conv_pass(pb_ref, C, wc_ref[1], bm_ref[2], st_res)
        conv_pass(act_ref, C, wc_ref[2], bm_ref[3], st_mid)
        conv_pass(pb_ref, C, wc_ref[3], bm_ref[4], st_res)
        conv_pass(feat_ref, C, wr_ref[...], br_ref[...], st_rec)

    feat, rec = pl.pallas_call(
        body,
        out_shape=(jax.ShapeDtypeStruct((B, C, L), bf16),
                   jax.ShapeDtypeStruct((B, CR, NPAD), bf16)),
        grid=(B,),
        in_specs=[
            pl.BlockSpec((None, Cc, H * W), lambda b: (b, 0, 0)),
            pl.BlockSpec((None, Cr, H * W), lambda b: (b, 0, 0)),
            pl.BlockSpec((MW, KW0), lambda b: (0, 0)),
            pl.BlockSpec((4, MW, KWC), lambda b: (0, 0, 0)),
            pl.BlockSpec((MW, KWC), lambda b: (0, 0)),
            pl.BlockSpec((5, C, 1), lambda b: (0, 0, 0)),
            pl.BlockSpec((C, 1), lambda b: (0, 0)),
        ],
        out_specs=(pl.BlockSpec((None, C, L), lambda b: (b, 0, 0)),
                   pl.BlockSpec((None, CR, NPAD), lambda b: (b, 0, 0))),
        scratch_shapes=[pltpu.VMEM((Cin0, L), bf16),
                        pltpu.VMEM((C, L), bf16),
                        pltpu.VMEM((C, L), bf16)],
        compiler_params=pltpu.CompilerParams(
            dimension_semantics=("parallel",),
            vmem_limit_bytes=100 << 20),
    )(ctf, ref_, w0m, wcm, wrm, bm, brm)

    ft = feat[:, :, _GL:_GL + Lp].reshape(B, C, Hp, _SL)[:, :, 1:H + 1, 1:W + 1]
    rc = rec[:, :, :Lp].reshape(B, CR, Hp, _SL)[:, :, 1:H + 1, 1:W + 1]
    return ft.astype(ctx.dtype), rc.astype(ctx.dtype)


# speedup vs baseline: 4.3822x; 1.0343x over previous
import jax
import jax.numpy as jnp
from jax import lax
from jax.experimental import pallas as pl
from jax.experimental.pallas import tpu as pltpu

_SLOPE = 0.01
_SL = 128
_TN = 896
_GL = 128


def kernel(w0, b0, w1a, b1a, w1b, b1b, w2a, b2a, w2b, b2b, wr, br, ctx, res):
    B, Cc, H, W = ctx.shape
    Cr = res.shape[1]
    Cin0 = Cc + Cr
    C = w0.shape[-1]
    CR = wr.shape[-1]
    Hp = H + 2
    Lp = Hp * _SL
    NT = 2 * (-(-Lp // (2 * _TN)))
    NPAD = NT * _TN
    L = _GL + NPAD + _GL
    KW0 = 3 * Cin0
    KWC = 3 * C
    MW = 3 * C
    f32 = jnp.float32
    bf16 = jnp.bfloat16

    ctf = ctx.reshape(B, Cc, H * W)
    ref_ = res.reshape(B, Cr, H * W)

    def _wT(w):
        cin, cout = w.shape[2], w.shape[3]
        wp = jnp.pad(w, ((0, 0), (0, 0), (0, 0), (0, C - cout)))
        return jnp.transpose(wp, (1, 3, 0, 2)).reshape(3 * C, 3 * cin)

    w0m = _wT(w0).astype(bf16)
    wcm = jnp.stack([_wT(w) for w in (w1a, w1b, w2a, w2b)]).astype(bf16)
    wrm = _wT(wr).astype(bf16)
    bm = jnp.stack([b.reshape(C, 1).astype(f32)
                    for b in (b0, b1a, b1b, b2a, b2b)])
    brm = jnp.pad(br, (0, C - CR)).reshape(C, 1).astype(f32)

    def body(ct_ref, rs_ref, w0_ref, wc_ref, wr_ref, bm_ref, br_ref,
             fout_ref, rout_ref, x_ref, feat_ref, rec_ref, act_ref, pb_ref):

        def lrelu(v):
            return jnp.where(v >= 0, v, _SLOPE * v)

        def interior(q0):
            l = q0 + lax.broadcasted_iota(jnp.int32, (1, _TN), 1)
            h = l >> 7
            w_ = l & 127
            return (h >= 1) & (h <= H) & (w_ >= 1) & (w_ <= W)

        def rolled_sum(p):
            p0, p1, p2 = p[0:C, :], p[C:2 * C, :], p[2 * C:3 * C, :]
            r0 = jnp.concatenate([p0[:, _TN - 1:], p0[:, :_TN - 1]], axis=1)
            r2 = jnp.concatenate([p2[:, 1:], p2[:, :1]], axis=1)
            return r0 + p1 + r2

        def tap_rhs(src_ref, cin, q0):
            v = src_ref[0:cin, pl.ds(_GL - _SL + q0, _TN + 2 * _SL)]
            return jnp.concatenate(
                [v[:, ky * _SL:ky * _SL + _TN] for ky in range(3)], axis=0)

        def conv_pass(src_ref, cin, w, bias, store):
            def two(i, carry):
                q0 = pl.multiple_of(i * (2 * _TN), 2 * _TN)
                p0 = jnp.dot(w, tap_rhs(src_ref, cin, q0),
                             preferred_element_type=f32)
                store(q0, rolled_sum(p0) + bias)
                p1 = jnp.dot(w, tap_rhs(src_ref, cin, q0 + _TN),
                             preferred_element_type=f32)
                store(q0 + _TN, rolled_sum(p1) + bias)
                return carry
            lax.fori_loop(0, NT // 2, two, 0)

        for ref in (feat_ref, act_ref, pb_ref):
            ref[:, 0:_GL] = jnp.zeros((C, _GL), bf16)
            ref[:, _GL + NPAD:L] = jnp.zeros((C, L - _GL - NPAD), bf16)

        x_ref[:, 0:_GL + _SL] = jnp.zeros((Cin0, _GL + _SL), bf16)
        x_ref[:, _GL + _SL + H * _SL:L] = (
            jnp.zeros((Cin0, L - _GL - _SL - H * _SL), bf16))
        RB = 4 * W
        DB = 4 * _SL

        def pad4(i, carry):
            so = pl.multiple_of(i * RB, 128)
            do = pl.multiple_of(i * DB, 128)
            vc = ct_ref[:, pl.ds(so, RB)].astype(bf16)
            vr = rs_ref[:, pl.ds(so, RB)].astype(bf16)

            def blk(v):
                return jnp.concatenate(
                    [jnp.pad(v[:, j * W:(j + 1) * W], ((0, 0), (1, _SL - W - 1)))
                     for j in range(4)], axis=1)

            x_ref[0:Cc, pl.ds(_GL + _SL + do, DB)] = blk(vc)
            x_ref[Cc:Cin0, pl.ds(_GL + _SL + do, DB)] = blk(vr)
            return carry
        lax.fori_loop(0, H // 4, pad4, 0)

        def st_head(q0, y):
            y = jnp.where(interior(q0), y, 0.0)
            feat_ref[:, pl.ds(_GL + q0, _TN)] = y.astype(bf16)
            act_ref[:, pl.ds(_GL + q0, _TN)] = lrelu(y).astype(bf16)

        def st_mid(q0, y):
            y = jnp.where(interior(q0), lrelu(y), 0.0)
            pb_ref[:, pl.ds(_GL + q0, _TN)] = y.astype(bf16)

        def st_res(q0, y):
            y = (jnp.where(interior(q0), y, 0.0)
                 + feat_ref[:, pl.ds(_GL + q0, _TN)].astype(f32))
            feat_ref[:, pl.ds(_GL + q0, _TN)] = y.astype(bf16)
            act_ref[:, pl.ds(_GL + q0, _TN)] = lrelu(y).astype(bf16)

        def st_rec(q0, y):
            rec_ref[0:CR, pl.ds(_GL + q0, _TN)] = y[0:CR, :].astype(bf16)

        def out4(i, carry):
            do = pl.multiple_of(i * DB, 128)
            so = pl.multiple_of(i * RB, 128)
            vf = feat_ref[:, pl.ds(_GL + _SL + do, DB)]
            vr = rec_ref[0:CR, pl.ds(_GL + _SL + do, DB)]

            def blk(v):
                return jnp.concatenate(
                    [v[:, j * _SL + 1:j * _SL + 1 + W] for j in range(4)],
                    axis=1).astype(f32)

            fout_ref[:, pl.ds(so, RB)] = blk(vf)
            rout_ref[0:CR, pl.ds(so, RB)] = blk(vr)
            return carry

        conv_pass(x_ref, Cin0, w0_ref[...], bm_ref[0], st_head)
        conv_pass(act_ref, C, wc_ref[0], bm_ref[1], st_mid)
        conv_pass(pb_ref, C, wc_ref[1], bm_ref[2], st_res)
        conv_pass(act_ref, C, wc_ref[2], bm_ref[3], st_mid)
        conv_pass(pb_ref, C, wc_ref[3], bm_ref[4], st_res)
        conv_pass(feat_ref, C, wr_ref[...], br_ref[...], st_rec)
        lax.fori_loop(0, H // 4, out4, 0)

    feat, rec = pl.pallas_call(
        body,
        out_shape=(jax.ShapeDtypeStruct((B, C, H * W), f32),
                   jax.ShapeDtypeStruct((B, CR, H * W), f32)),
        grid=(B,),
        in_specs=[
            pl.BlockSpec((None, Cc, H * W), lambda b: (b, 0, 0)),
            pl.BlockSpec((None, Cr, H * W), lambda b: (b, 0, 0)),
            pl.BlockSpec((MW, KW0), lambda b: (0, 0)),
            pl.BlockSpec((4, MW, KWC), lambda b: (0, 0, 0)),
            pl.BlockSpec((MW, KWC), lambda b: (0, 0)),
            pl.BlockSpec((5, C, 1), lambda b: (0, 0, 0)),
            pl.BlockSpec((C, 1), lambda b: (0, 0)),
        ],
        out_specs=(pl.BlockSpec((None, C, H * W), lambda b: (b, 0, 0)),
                   pl.BlockSpec((None, CR, H * W), lambda b: (b, 0, 0))),
        scratch_shapes=[pltpu.VMEM((Cin0, L), bf16),
                        pltpu.VMEM((C, L), bf16),
                        pltpu.VMEM((8, L), bf16),
                        pltpu.VMEM((C, L), bf16),
                        pltpu.VMEM((C, L), bf16)],
        compiler_params=pltpu.CompilerParams(
            dimension_semantics=("parallel",),
            vmem_limit_bytes=100 << 20),
    )(ctf, ref_, w0m, wcm, wrm, bm, brm)

    return (feat.reshape(B, C, H, W).astype(ctx.dtype),
            rec.reshape(B, CR, H, W).astype(ctx.dtype))


# speedup vs baseline: 5.0002x; 1.1410x over previous
import jax
import jax.numpy as jnp
from jax import lax
from jax.experimental import pallas as pl
from jax.experimental.pallas import tpu as pltpu

_SLOPE = 0.01
_SL = 128
_TN = 896
_GL = 128


def kernel(w0, b0, w1a, b1a, w1b, b1b, w2a, b2a, w2b, b2b, wr, br, ctx, res):
    B, Cc, H, W = ctx.shape
    Cr = res.shape[1]
    Cin0 = Cc + Cr
    C = w0.shape[-1]
    CR = wr.shape[-1]
    Hp = H + 2
    Lp = Hp * _SL
    NT = 2 * (-(-Lp // (2 * _TN)))
    NPAD = NT * _TN
    L = _GL + NPAD + _GL
    KW0 = 3 * Cin0
    KWC = 3 * C
    MW = 3 * C
    f32 = jnp.float32
    bf16 = jnp.bfloat16

    ctf = ctx.reshape(B, Cc, H * W)
    ref_ = res.reshape(B, Cr, H * W)

    def _wT(w):
        cin, cout = w.shape[2], w.shape[3]
        wp = jnp.pad(w, ((0, 0), (0, 0), (0, 0), (0, C - cout)))
        return jnp.transpose(wp, (1, 3, 0, 2)).reshape(3 * C, 3 * cin)

    w0m = _wT(w0).astype(bf16)
    wcm = jnp.stack([_wT(w) for w in (w1a, w1b, w2a, w2b)]).astype(bf16)
    wrm = _wT(wr).astype(bf16)
    bm = jnp.stack([b.reshape(C, 1).astype(f32)
                    for b in (b0, b1a, b1b, b2a, b2b)])
    brm = jnp.pad(br, (0, C - CR)).reshape(C, 1).astype(f32)

    def body(ct_ref, rs_ref, w0_ref, wc_ref, wr_ref, bm_ref, br_ref,
             fout_ref, rout_ref, x_ref, feat_ref, rec_ref, act_ref, pb_ref):

        def lrelu(v):
            return jnp.where(v >= 0, v, _SLOPE * v)

        def interior(q0):
            l = q0 + lax.broadcasted_iota(jnp.int32, (1, _TN), 1)
            h = l >> 7
            w_ = l & 127
            return (h >= 1) & (h <= H) & (w_ >= 1) & (w_ <= W)

        def rolled_sum(p):
            p0, p1, p2 = p[0:C, :], p[C:2 * C, :], p[2 * C:3 * C, :]
            r0 = jnp.concatenate([p0[:, _TN - 1:], p0[:, :_TN - 1]], axis=1)
            r2 = jnp.concatenate([p2[:, 1:], p2[:, :1]], axis=1)
            return r0 + p1 + r2

        def tap_rhs(src_ref, cin, q0):
            v = src_ref[0:cin, pl.ds(_GL - _SL + q0, _TN + 2 * _SL)]
            return jnp.concatenate(
                [v[:, ky * _SL:ky * _SL + _TN] for ky in range(3)], axis=0)

        def conv_pass(src_ref, cin, w, bias, store):
            def tile(q0):
                pv = jnp.dot(w, tap_rhs(src_ref, cin, q0),
                             preferred_element_type=f32)
                store(q0, rolled_sum(pv) + bias)

            def quad(i, carry):
                q0 = pl.multiple_of(i * (4 * _TN), 4 * _TN)
                for j in range(4):
                    tile(q0 + j * _TN)
                return carry
            lax.fori_loop(0, NT // 4, quad, 0)
            for j in range(NT - 4 * (NT // 4)):
                tile((4 * (NT // 4) + j) * _TN)

        for ref in (feat_ref, act_ref, pb_ref):
            ref[:, 0:_GL] = jnp.zeros((C, _GL), bf16)
            ref[:, _GL + NPAD:L] = jnp.zeros((C, L - _GL - NPAD), bf16)

        x_ref[:, 0:_GL + _SL] = jnp.zeros((Cin0, _GL + _SL), bf16)
        x_ref[:, _GL + _SL + H * _SL:L] = (
            jnp.zeros((Cin0, L - _GL - _SL - H * _SL), bf16))
        RB = 4 * W
        DB = 4 * _SL

        def pad4(i, carry):
            so = pl.multiple_of(i * RB, 128)
            do = pl.multiple_of(i * DB, 128)
            vc = ct_ref[:, pl.ds(so, RB)].astype(bf16)
            vr = rs_ref[:, pl.ds(so, RB)].astype(bf16)

            def blk(v):
                return jnp.concatenate(
                    [jnp.pad(v[:, j * W:(j + 1) * W], ((0, 0), (1, _SL - W - 1)))
                     for j in range(4)], axis=1)

            x_ref[0:Cc, pl.ds(_GL + _SL + do, DB)] = blk(vc)
            x_ref[Cc:Cin0, pl.ds(_GL + _SL + do, DB)] = blk(vr)
            return carry
        lax.fori_loop(0, H // 4, pad4, 0)

        def st_head(q0, y):
            y = jnp.where(interior(q0), y, 0.0)
            feat_ref[:, pl.ds(_GL + q0, _TN)] = y.astype(bf16)
            act_ref[:, pl.ds(_GL + q0, _TN)] = lrelu(y).astype(bf16)

        def st_mid(q0, y):
            y = jnp.where(interior(q0), lrelu(y), 0.0)
            pb_ref[:, pl.ds(_GL + q0, _TN)] = y.astype(bf16)

        def st_res(q0, y):
            y = (jnp.where(interior(q0), y, 0.0)
                 + feat_ref[:, pl.ds(_GL + q0, _TN)].astype(f32))
            feat_ref[:, pl.ds(_GL + q0, _TN)] = y.astype(bf16)
            act_ref[:, pl.ds(_GL + q0, _TN)] = lrelu(y).astype(bf16)

        def st_rec(q0, y):
            rec_ref[0:CR, pl.ds(_GL + q0, _TN)] = y[0:CR, :].astype(bf16)

        def out4(i, carry):
            do = pl.multiple_of(i * DB, 128)
            so = pl.multiple_of(i * RB, 128)
            vf = feat_ref[:, pl.ds(_GL + _SL + do, DB)]
            vr = rec_ref[0:CR, pl.ds(_GL + _SL + do, DB)]

            def blk(v):
                return jnp.concatenate(
                    [v[:, j * _SL + 1:j * _SL + 1 + W] for j in range(4)],
                    axis=1).astype(f32)

            fout_ref[:, pl.ds(so, RB)] = blk(vf)
            rout_ref[0:CR, pl.ds(so, RB)] = blk(vr)
            return carry

        conv_pass(x_ref, Cin0, w0_ref[...], bm_ref[0], st_head)
        conv_pass(act_ref, C, wc_ref[0], bm_ref[1], st_mid)
        conv_pass(pb_ref, C, wc_ref[1], bm_ref[2], st_res)
        conv_pass(act_ref, C, wc_ref[2], bm_ref[3], st_mid)
        conv_pass(pb_ref, C, wc_ref[3], bm_ref[4], st_res)
        conv_pass(feat_ref, C, wr_ref[...], br_ref[...], st_rec)
        lax.fori_loop(0, H // 4, out4, 0)

    feat, rec = pl.pallas_call(
        body,
        out_shape=(jax.ShapeDtypeStruct((B, C, H * W), f32),
                   jax.ShapeDtypeStruct((B, CR, H * W), f32)),
        grid=(B,),
        in_specs=[
            pl.BlockSpec((None, Cc, H * W), lambda b: (b, 0, 0)),
            pl.BlockSpec((None, Cr, H * W), lambda b: (b, 0, 0)),
            pl.BlockSpec((MW, KW0), lambda b: (0, 0)),
            pl.BlockSpec((4, MW, KWC), lambda b: (0, 0, 0)),
            pl.BlockSpec((MW, KWC), lambda b: (0, 0)),
            pl.BlockSpec((5, C, 1), lambda b: (0, 0, 0)),
            pl.BlockSpec((C, 1), lambda b: (0, 0)),
        ],
        out_specs=(pl.BlockSpec((None, C, H * W), lambda b: (b, 0, 0)),
                   pl.BlockSpec((None, CR, H * W), lambda b: (b, 0, 0))),
        scratch_shapes=[pltpu.VMEM((Cin0, L), bf16),
                        pltpu.VMEM((C, L), bf16),
                        pltpu.VMEM((8, L), bf16),
                        pltpu.VMEM((C, L), bf16),
                        pltpu.VMEM((C, L), bf16)],
        compiler_params=pltpu.CompilerParams(
            dimension_semantics=("parallel",),
            vmem_limit_bytes=100 << 20),
    )(ctf, ref_, w0m, wcm, wrm, bm, brm)

    return (feat.reshape(B, C, H, W).astype(ctx.dtype),
            rec.reshape(B, CR, H, W).astype(ctx.dtype))


# speedup vs baseline: 5.4260x; 1.0851x over previous
import jax
import jax.numpy as jnp
from jax import lax
from jax.experimental import pallas as pl
from jax.experimental.pallas import tpu as pltpu

_SLOPE = 0.01
_SL = 128
_TN = 896
_GL = 128


def kernel(w0, b0, w1a, b1a, w1b, b1b, w2a, b2a, w2b, b2b, wr, br, ctx, res):
    B, Cc, H, W = ctx.shape
    Cr = res.shape[1]
    Cin0 = Cc + Cr
    C = w0.shape[-1]
    CR = wr.shape[-1]
    Hp = H + 2
    Lp = Hp * _SL
    NT = 2 * (-(-Lp // (2 * _TN)))
    NPAD = NT * _TN
    L = _GL + NPAD + _GL
    KW0 = 3 * Cin0
    KWC = 3 * C
    MW = 3 * C
    f32 = jnp.float32
    bf16 = jnp.bfloat16

    ctf = ctx.reshape(B, Cc, H * W)
    ref_ = res.reshape(B, Cr, H * W)

    def _wT(w):
        cin, cout = w.shape[2], w.shape[3]
        wp = jnp.pad(w, ((0, 0), (0, 0), (0, 0), (0, C - cout)))
        return jnp.transpose(wp, (1, 3, 0, 2)).reshape(3 * C, 3 * cin)

    w0m = _wT(w0).astype(bf16)
    wcm = jnp.stack([_wT(w) for w in (w1a, w1b, w2a, w2b)]).astype(bf16)
    wrm = _wT(wr).astype(bf16)
    bm = jnp.stack([b.reshape(C, 1).astype(f32)
                    for b in (b0, b1a, b1b, b2a, b2b)])
    brm = jnp.pad(br, (0, C - CR)).reshape(C, 1).astype(f32)

    def body(ct_ref, rs_ref, w0_ref, wc_ref, wr_ref, bm_ref, br_ref,
             fout_ref, rout_ref, x_ref, feat_ref, rec_ref, act_ref, pb_ref):

        def lrelu(v):
            return jnp.where(v >= 0, v, _SLOPE * v)

        def interior(q0):
            l = q0 + lax.broadcasted_iota(jnp.int32, (1, _TN), 1)
            h = l >> 7
            w_ = l & 127
            return (h >= 1) & (h <= H) & (w_ >= 1) & (w_ <= W)

        def rolled_sum(p):
            p0, p1, p2 = p[0:C, :], p[C:2 * C, :], p[2 * C:3 * C, :]
            r0 = jnp.concatenate([p0[:, _TN - 1:], p0[:, :_TN - 1]], axis=1)
            r2 = jnp.concatenate([p2[:, 1:], p2[:, :1]], axis=1)
            return r0 + p1 + r2

        def tap_rhs(src_ref, cin, q0):
            v = src_ref[0:cin, pl.ds(_GL - _SL + q0, _TN + 2 * _SL)]
            return jnp.concatenate(
                [v[:, ky * _SL:ky * _SL + _TN] for ky in range(3)], axis=0)

        def conv_pass(src_ref, cin, w, bias, store):
            def tile(q0):
                pv = jnp.dot(w, tap_rhs(src_ref, cin, q0),
                             preferred_element_type=f32)
                store(q0, rolled_sum(pv) + bias)

            def seven(i, carry):
                q0 = pl.multiple_of(i * (7 * _TN), 7 * _TN)
                for j in range(7):
                    tile(q0 + j * _TN)
                return carry
            lax.fori_loop(0, NT // 7, seven, 0)
            for j in range(NT - 7 * (NT // 7)):
                tile((7 * (NT // 7) + j) * _TN)

        for ref in (feat_ref, act_ref, pb_ref):
            ref[:, 0:_GL] = jnp.zeros((C, _GL), bf16)
            ref[:, _GL + NPAD:L] = jnp.zeros((C, L - _GL - NPAD), bf16)

        x_ref[:, 0:_GL + _SL] = jnp.zeros((Cin0, _GL + _SL), bf16)
        x_ref[:, _GL + _SL + H * _SL:L] = (
            jnp.zeros((Cin0, L - _GL - _SL - H * _SL), bf16))
        RB = 4 * W
        DB = 4 * _SL

        def pad4(i, carry):
            so = pl.multiple_of(i * RB, 128)
            do = pl.multiple_of(i * DB, 128)
            vc = ct_ref[:, pl.ds(so, RB)].astype(bf16)
            vr = rs_ref[:, pl.ds(so, RB)].astype(bf16)

            def blk(v):
                return jnp.concatenate(
                    [jnp.pad(v[:, j * W:(j + 1) * W], ((0, 0), (1, _SL - W - 1)))
                     for j in range(4)], axis=1)

            x_ref[0:Cc, pl.ds(_GL + _SL + do, DB)] = blk(vc)
            x_ref[Cc:Cin0, pl.ds(_GL + _SL + do, DB)] = blk(vr)
            return carry
        lax.fori_loop(0, H // 4, pad4, 0)

        def st_head(q0, y):
            y = jnp.where(interior(q0), y, 0.0)
            feat_ref[:, pl.ds(_GL + q0, _TN)] = y.astype(bf16)
            act_ref[:, pl.ds(_GL + q0, _TN)] = lrelu(y).astype(bf16)

        def st_mid(q0, y):
            y = jnp.where(interior(q0), lrelu(y), 0.0)
            pb_ref[:, pl.ds(_GL + q0, _TN)] = y.astype(bf16)

        def st_res(q0, y):
            y = (jnp.where(interior(q0), y, 0.0)
                 + feat_ref[:, pl.ds(_GL + q0, _TN)].astype(f32))
            feat_ref[:, pl.ds(_GL + q0, _TN)] = y.astype(bf16)
            act_ref[:, pl.ds(_GL + q0, _TN)] = lrelu(y).astype(bf16)

        def st_rec(q0, y):
            rec_ref[0:CR, pl.ds(_GL + q0, _TN)] = y[0:CR, :].astype(bf16)

        def out4(i, carry):
            do = pl.multiple_of(i * DB, 128)
            so = pl.multiple_of(i * RB, 128)
            vf = feat_ref[:, pl.ds(_GL + _SL + do, DB)]
            vr = rec_ref[0:CR, pl.ds(_GL + _SL + do, DB)]

            def blk(v):
                return jnp.concatenate(
                    [v[:, j * _SL + 1:j * _SL + 1 + W] for j in range(4)],
                    axis=1).astype(f32)

            fout_ref[:, pl.ds(so, RB)] = blk(vf)
            rout_ref[0:CR, pl.ds(so, RB)] = blk(vr)
            return carry

        conv_pass(x_ref, Cin0, w0_ref[...], bm_ref[0], st_head)
        conv_pass(act_ref, C, wc_ref[0], bm_ref[1], st_mid)
        conv_pass(pb_ref, C, wc_ref[1], bm_ref[2], st_res)
        conv_pass(act_ref, C, wc_ref[2], bm_ref[3], st_mid)
        conv_pass(pb_ref, C, wc_ref[3], bm_ref[4], st_res)
        conv_pass(feat_ref, C, wr_ref[...], br_ref[...], st_rec)
        lax.fori_loop(0, H // 4, out4, 0)

    feat, rec = pl.pallas_call(
        body,
        out_shape=(jax.ShapeDtypeStruct((B, C, H * W), f32),
                   jax.ShapeDtypeStruct((B, CR, H * W), f32)),
        grid=(B,),
        in_specs=[
            pl.BlockSpec((None, Cc, H * W), lambda b: (b, 0, 0)),
            pl.BlockSpec((None, Cr, H * W), lambda b: (b, 0, 0)),
            pl.BlockSpec((MW, KW0), lambda b: (0, 0)),
            pl.BlockSpec((4, MW, KWC), lambda b: (0, 0, 0)),
            pl.BlockSpec((MW, KWC), lambda b: (0, 0)),
            pl.BlockSpec((5, C, 1), lambda b: (0, 0, 0)),
            pl.BlockSpec((C, 1), lambda b: (0, 0)),
        ],
        out_specs=(pl.BlockSpec((None, C, H * W), lambda b: (b, 0, 0)),
                   pl.BlockSpec((None, CR, H * W), lambda b: (b, 0, 0))),
        scratch_shapes=[pltpu.VMEM((Cin0, L), bf16),
                        pltpu.VMEM((C, L), bf16),
                        pltpu.VMEM((8, L), bf16),
                        pltpu.VMEM((C, L), bf16),
                        pltpu.VMEM((C, L), bf16)],
        compiler_params=pltpu.CompilerParams(
            dimension_semantics=("parallel",),
            vmem_limit_bytes=100 << 20),
    )(ctf, ref_, w0m, wcm, wrm, bm, brm)

    return (feat.reshape(B, C, H, W).astype(ctx.dtype),
            rec.reshape(B, CR, H, W).astype(ctx.dtype))


# speedup vs baseline: 6.2737x; 1.1562x over previous
import jax
import jax.numpy as jnp
from jax import lax
from jax.experimental import pallas as pl
from jax.experimental.pallas import tpu as pltpu

_SLOPE = 0.01
_SL = 128
_TN = 896
_GL = 128


def kernel(w0, b0, w1a, b1a, w1b, b1b, w2a, b2a, w2b, b2b, wr, br, ctx, res):
    B, Cc, H, W = ctx.shape
    Cr = res.shape[1]
    Cin0 = Cc + Cr
    C = w0.shape[-1]
    CR = wr.shape[-1]
    Hp = H + 2
    Lp = Hp * _SL
    NT = 2 * (-(-Lp // (2 * _TN)))
    NPAD = NT * _TN
    L = _GL + NPAD + _GL
    KW0 = 3 * Cin0
    KWC = 3 * C
    MW = 3 * C
    f32 = jnp.float32
    bf16 = jnp.bfloat16

    ctf = ctx.reshape(B, Cc, H * W)
    ref_ = res.reshape(B, Cr, H * W)

    def _wT(w):
        cin, cout = w.shape[2], w.shape[3]
        wp = jnp.pad(w, ((0, 0), (0, 0), (0, 0), (0, C - cout)))
        return jnp.transpose(wp, (1, 3, 0, 2)).reshape(3 * C, 3 * cin)

    w0m = _wT(w0).astype(bf16)
    wcm = jnp.stack([_wT(w) for w in (w1a, w1b, w2a, w2b)]).astype(bf16)
    wrm = _wT(wr).astype(bf16)
    bm = jnp.stack([b.reshape(C, 1).astype(f32)
                    for b in (b0, b1a, b1b, b2a, b2b)])
    brm = jnp.pad(br, (0, C - CR)).reshape(C, 1).astype(f32)

    def body(ct_ref, rs_ref, w0_ref, wc_ref, wr_ref, bm_ref, br_ref,
             fout_ref, rout_ref, x_ref, feat_ref, rec_ref, act_ref, pb_ref):

        def lrelu(v):
            return jnp.where(v >= 0, v, _SLOPE * v)

        def interior(q0):
            l = q0 + lax.broadcasted_iota(jnp.int32, (1, _TN), 1)
            h = l >> 7
            w_ = l & 127
            return (h >= 1) & (h <= H) & (w_ >= 1) & (w_ <= W)

        def rolled_sum(p):
            p0, p1, p2 = p[0:C, :], p[C:2 * C, :], p[2 * C:3 * C, :]
            r0 = jnp.concatenate([p0[:, _TN - 1:], p0[:, :_TN - 1]], axis=1)
            r2 = jnp.concatenate([p2[:, 1:], p2[:, :1]], axis=1)
            return r0 + p1 + r2

        def tap_rhs(src_ref, cin, q0):
            v = src_ref[0:cin, pl.ds(_GL - _SL + q0, _TN + 2 * _SL)]
            return jnp.concatenate(
                [v[:, ky * _SL:ky * _SL + _TN] for ky in range(3)], axis=0)

        def conv_pass(src_ref, cin, w, bias, store):
            def tile(q0):
                pv = jnp.dot(w, tap_rhs(src_ref, cin, q0),
                             preferred_element_type=f32)
                store(q0, rolled_sum(pv) + bias)

            for j in range(NT):
                tile(j * _TN)

        for ref in (feat_ref, act_ref, pb_ref):
            ref[:, 0:_GL] = jnp.zeros((C, _GL), bf16)
            ref[:, _GL + NPAD:L] = jnp.zeros((C, L - _GL - NPAD), bf16)

        x_ref[:, 0:_GL + _SL] = jnp.zeros((Cin0, _GL + _SL), bf16)
        x_ref[:, _GL + _SL + H * _SL:L] = (
            jnp.zeros((Cin0, L - _GL - _SL - H * _SL), bf16))
        RB = 4 * W
        DB = 4 * _SL

        def pad4(i, carry):
            so = pl.multiple_of(i * RB, 128)
            do = pl.multiple_of(i * DB, 128)
            vc = ct_ref[:, pl.ds(so, RB)].astype(bf16)
            vr = rs_ref[:, pl.ds(so, RB)].astype(bf16)

            def blk(v):
                return jnp.concatenate(
                    [jnp.pad(v[:, j * W:(j + 1) * W], ((0, 0), (1, _SL - W - 1)))
                     for j in range(4)], axis=1)

            x_ref[0:Cc, pl.ds(_GL + _SL + do, DB)] = blk(vc)
            x_ref[Cc:Cin0, pl.ds(_GL + _SL + do, DB)] = blk(vr)
            return carry
        lax.fori_loop(0, H // 4, pad4, 0)

        def st_head(q0, y):
            y = jnp.where(interior(q0), y, 0.0)
            feat_ref[:, pl.ds(_GL + q0, _TN)] = y.astype(bf16)
            act_ref[:, pl.ds(_GL + q0, _TN)] = lrelu(y).astype(bf16)

        def st_mid(q0, y):
            y = jnp.where(interior(q0), lrelu(y), 0.0)
            pb_ref[:, pl.ds(_GL + q0, _TN)] = y.astype(bf16)

        def st_res(q0, y):
            y = (jnp.where(interior(q0), y, 0.0)
                 + feat_ref[:, pl.ds(_GL + q0, _TN)].astype(f32))
            feat_ref[:, pl.ds(_GL + q0, _TN)] = y.astype(bf16)
            act_ref[:, pl.ds(_GL + q0, _TN)] = lrelu(y).astype(bf16)

        def st_rec(q0, y):
            rec_ref[0:CR, pl.ds(_GL + q0, _TN)] = y[0:CR, :].astype(bf16)

        def out4(i, carry):
            do = pl.multiple_of(i * DB, 128)
            so = pl.multiple_of(i * RB, 128)
            vf = feat_ref[:, pl.ds(_GL + _SL + do, DB)]
            vr = rec_ref[0:CR, pl.ds(_GL + _SL + do, DB)]

            def blk(v):
                return jnp.concatenate(
                    [v[:, j * _SL + 1:j * _SL + 1 + W] for j in range(4)],
                    axis=1).astype(f32)

            fout_ref[:, pl.ds(so, RB)] = blk(vf)
            rout_ref[0:CR, pl.ds(so, RB)] = blk(vr)
            return carry

        conv_pass(x_ref, Cin0, w0_ref[...], bm_ref[0], st_head)
        conv_pass(act_ref, C, wc_ref[0], bm_ref[1], st_mid)
        conv_pass(pb_ref, C, wc_ref[1], bm_ref[2], st_res)
        conv_pass(act_ref, C, wc_ref[2], bm_ref[3], st_mid)
        conv_pass(pb_ref, C, wc_ref[3], bm_ref[4], st_res)
        conv_pass(feat_ref, C, wr_ref[...], br_ref[...], st_rec)
        lax.fori_loop(0, H // 4, out4, 0)

    feat, rec = pl.pallas_call(
        body,
        out_shape=(jax.ShapeDtypeStruct((B, C, H * W), f32),
                   jax.ShapeDtypeStruct((B, CR, H * W), f32)),
        grid=(B,),
        in_specs=[
            pl.BlockSpec((None, Cc, H * W), lambda b: (b, 0, 0)),
            pl.BlockSpec((None, Cr, H * W), lambda b: (b, 0, 0)),
            pl.BlockSpec((MW, KW0), lambda b: (0, 0)),
            pl.BlockSpec((4, MW, KWC), lambda b: (0, 0, 0)),
            pl.BlockSpec((MW, KWC), lambda b: (0, 0)),
            pl.BlockSpec((5, C, 1), lambda b: (0, 0, 0)),
            pl.BlockSpec((C, 1), lambda b: (0, 0)),
        ],
        out_specs=(pl.BlockSpec((None, C, H * W), lambda b: (b, 0, 0)),
                   pl.BlockSpec((None, CR, H * W), lambda b: (b, 0, 0))),
        scratch_shapes=[pltpu.VMEM((Cin0, L), bf16),
                        pltpu.VMEM((C, L), bf16),
                        pltpu.VMEM((8, L), bf16),
                        pltpu.VMEM((C, L), bf16),
                        pltpu.VMEM((C, L), bf16)],
        compiler_params=pltpu.CompilerParams(
            dimension_semantics=("parallel",),
            vmem_limit_bytes=100 << 20),
    )(ctf, ref_, w0m, wcm, wrm, bm, brm)

    return (feat.reshape(B, C, H, W).astype(ctx.dtype),
            rec.reshape(B, CR, H, W).astype(ctx.dtype))


# speedup vs baseline: 6.3211x; 1.0076x over previous
import jax
import jax.numpy as jnp
from jax import lax
from jax.experimental import pallas as pl
from jax.experimental.pallas import tpu as pltpu

_SLOPE = 0.01
_SL = 128
_TN = 896
_GL = 128


def kernel(w0, b0, w1a, b1a, w1b, b1b, w2a, b2a, w2b, b2b, wr, br, ctx, res):
    B, Cc, H, W = ctx.shape
    Cr = res.shape[1]
    Cin0 = Cc + Cr
    C = w0.shape[-1]
    CR = wr.shape[-1]
    Hp = H + 2
    Lp = Hp * _SL
    NT = 2 * (-(-Lp // (2 * _TN)))
    NPAD = NT * _TN
    L = _GL + NPAD + _GL
    KW0 = 3 * Cin0
    KWC = 3 * C
    MW = 3 * C
    f32 = jnp.float32
    bf16 = jnp.bfloat16

    ctf = ctx.reshape(B, Cc, H * W)
    ref_ = res.reshape(B, Cr, H * W)

    def _wT(w, gs):
        cin, cout = w.shape[2], w.shape[3]
        wp = jnp.pad(w, ((0, 0), (0, 0), (0, 0), (0, gs - cout)))
        return jnp.transpose(wp, (1, 3, 0, 2)).reshape(3 * gs, 3 * cin)

    GR = 8
    w0m = _wT(w0, C).astype(bf16)
    wcm = jnp.stack([_wT(w, C) for w in (w1a, w1b, w2a, w2b)]).astype(bf16)
    wrm = _wT(wr, GR).astype(bf16)
    bm = jnp.stack([b.reshape(C, 1).astype(f32)
                    for b in (b0, b1a, b1b, b2a, b2b)])
    brm = jnp.pad(br, (0, GR - CR)).reshape(GR, 1).astype(f32)

    def body(ct_ref, rs_ref, w0_ref, wc_ref, wr_ref, bm_ref, br_ref,
             fout_ref, rout_ref, x_ref, feat_ref, rec_ref, act_ref, pb_ref):

        def lrelu(v):
            return jnp.where(v >= 0, v, _SLOPE * v)

        def interior(q0):
            l = q0 + lax.broadcasted_iota(jnp.int32, (1, _TN), 1)
            h = l >> 7
            w_ = l & 127
            return (h >= 1) & (h <= H) & (w_ >= 1) & (w_ <= W)

        def rolled_sum(p, gs):
            p0, p1, p2 = p[0:gs, :], p[gs:2 * gs, :], p[2 * gs:3 * gs, :]
            r0 = jnp.concatenate([p0[:, _TN - 1:], p0[:, :_TN - 1]], axis=1)
            r2 = jnp.concatenate([p2[:, 1:], p2[:, :1]], axis=1)
            return r0 + p1 + r2

        def tap_rhs(src_ref, cin, q0):
            v = src_ref[0:cin, pl.ds(_GL - _SL + q0, _TN + 2 * _SL)]
            return jnp.concatenate(
                [v[:, ky * _SL:ky * _SL + _TN] for ky in range(3)], axis=0)

        def conv_pass(src_ref, cin, w, bias, store, gs=C):
            def tile(q0):
                pv = jnp.dot(w, tap_rhs(src_ref, cin, q0),
                             preferred_element_type=f32)
                store(q0, rolled_sum(pv, gs) + bias)

            for j in range(NT):
                tile(j * _TN)

        for ref in (feat_ref, act_ref, pb_ref):
            ref[:, 0:_GL] = jnp.zeros((C, _GL), bf16)
            ref[:, _GL + NPAD:L] = jnp.zeros((C, L - _GL - NPAD), bf16)

        x_ref[:, 0:_GL + _SL] = jnp.zeros((Cin0, _GL + _SL), bf16)
        x_ref[:, _GL + _SL + H * _SL:L] = (
            jnp.zeros((Cin0, L - _GL - _SL - H * _SL), bf16))
        RB = 4 * W
        DB = 4 * _SL

        def pad4(i, carry):
            so = pl.multiple_of(i * RB, 128)
            do = pl.multiple_of(i * DB, 128)
            vc = ct_ref[:, pl.ds(so, RB)].astype(bf16)
            vr = rs_ref[:, pl.ds(so, RB)].astype(bf16)

            def blk(v):
                return jnp.concatenate(
                    [jnp.pad(v[:, j * W:(j + 1) * W], ((0, 0), (1, _SL - W - 1)))
                     for j in range(4)], axis=1)

            x_ref[0:Cc, pl.ds(_GL + _SL + do, DB)] = blk(vc)
            x_ref[Cc:Cin0, pl.ds(_GL + _SL + do, DB)] = blk(vr)
            return carry
        lax.fori_loop(0, H // 4, pad4, 0)

        def st_head(q0, y):
            y = jnp.where(interior(q0), y, 0.0)
            feat_ref[:, pl.ds(_GL + q0, _TN)] = y.astype(bf16)
            act_ref[:, pl.ds(_GL + q0, _TN)] = lrelu(y).astype(bf16)

        def st_mid(q0, y):
            y = jnp.where(interior(q0), lrelu(y), 0.0)
            pb_ref[:, pl.ds(_GL + q0, _TN)] = y.astype(bf16)

        def st_res(q0, y):
            y = (jnp.where(interior(q0), y, 0.0)
                 + feat_ref[:, pl.ds(_GL + q0, _TN)].astype(f32))
            feat_ref[:, pl.ds(_GL + q0, _TN)] = y.astype(bf16)
            act_ref[:, pl.ds(_GL + q0, _TN)] = lrelu(y).astype(bf16)

        def st_rec(q0, y):
            rec_ref[0:CR, pl.ds(_GL + q0, _TN)] = y[0:CR, :].astype(bf16)

        def out4(i, carry):
            do = pl.multiple_of(i * DB, 128)
            so = pl.multiple_of(i * RB, 128)
            vf = feat_ref[:, pl.ds(_GL + _SL + do, DB)]
            vr = rec_ref[0:CR, pl.ds(_GL + _SL + do, DB)]

            def blk(v):
                return jnp.concatenate(
                    [v[:, j * _SL + 1:j * _SL + 1 + W] for j in range(4)],
                    axis=1).astype(f32)

            fout_ref[:, pl.ds(so, RB)] = blk(vf)
            rout_ref[0:CR, pl.ds(so, RB)] = blk(vr)
            return carry

        conv_pass(x_ref, Cin0, w0_ref[...], bm_ref[0], st_head)
        conv_pass(act_ref, C, wc_ref[0], bm_ref[1], st_mid)
        conv_pass(pb_ref, C, wc_ref[1], bm_ref[2], st_res)
        conv_pass(act_ref, C, wc_ref[2], bm_ref[3], st_mid)
        conv_pass(pb_ref, C, wc_ref[3], bm_ref[4], st_res)
        conv_pass(feat_ref, C, wr_ref[...], br_ref[...], st_rec, gs=GR)
        lax.fori_loop(0, H // 4, out4, 0)

    feat, rec = pl.pallas_call(
        body,
        out_shape=(jax.ShapeDtypeStruct((B, C, H * W), f32),
                   jax.ShapeDtypeStruct((B, CR, H * W), f32)),
        grid=(B,),
        in_specs=[
            pl.BlockSpec((None, Cc, H * W), lambda b: (b, 0, 0)),
            pl.BlockSpec((None, Cr, H * W), lambda b: (b, 0, 0)),
            pl.BlockSpec((MW, KW0), lambda b: (0, 0)),
            pl.BlockSpec((4, MW, KWC), lambda b: (0, 0, 0)),
            pl.BlockSpec((3 * GR, KWC), lambda b: (0, 0)),
            pl.BlockSpec((5, C, 1), lambda b: (0, 0, 0)),
            pl.BlockSpec((GR, 1), lambda b: (0, 0)),
        ],
        out_specs=(pl.BlockSpec((None, C, H * W), lambda b: (b, 0, 0)),
                   pl.BlockSpec((None, CR, H * W), lambda b: (b, 0, 0))),
        scratch_shapes=[pltpu.VMEM((Cin0, L), bf16),
                        pltpu.VMEM((C, L), bf16),
                        pltpu.VMEM((8, L), bf16),
                        pltpu.VMEM((C, L), bf16),
                        pltpu.VMEM((C, L), bf16)],
        compiler_params=pltpu.CompilerParams(
            dimension_semantics=("parallel",),
            vmem_limit_bytes=100 << 20),
    )(ctf, ref_, w0m, wcm, wrm, bm, brm)

    return (feat.reshape(B, C, H, W).astype(ctx.dtype),
            rec.reshape(B, CR, H, W).astype(ctx.dtype))


# speedup vs baseline: 6.9289x; 1.0962x over previous
import jax
import jax.numpy as jnp
from jax import lax
from jax.experimental import pallas as pl
from jax.experimental.pallas import tpu as pltpu

_SLOPE = 0.01
_SL = 128
_TN = 896
_GL = 128


def kernel(w0, b0, w1a, b1a, w1b, b1b, w2a, b2a, w2b, b2b, wr, br, ctx, res):
    B, Cc, H, W = ctx.shape
    Cr = res.shape[1]
    Cin0 = Cc + Cr
    C = w0.shape[-1]
    CR = wr.shape[-1]
    Hp = H + 2
    Lp = Hp * _SL
    NT = 2 * (-(-Lp // (2 * _TN)))
    NPAD = NT * _TN
    L = _GL + NPAD + _GL
    KW0 = 3 * Cin0
    KWC = 3 * C
    MW = 3 * C
    f32 = jnp.float32
    bf16 = jnp.bfloat16

    ctf = ctx.reshape(B, Cc, H * W)
    ref_ = res.reshape(B, Cr, H * W)

    def _wT(w, gs):
        cin, cout = w.shape[2], w.shape[3]
        wp = jnp.pad(w, ((0, 0), (0, 0), (0, 0), (0, gs - cout)))
        return jnp.transpose(wp, (1, 3, 0, 2)).reshape(3 * gs, 3 * cin)

    GR = 8
    w0m = _wT(w0, C).astype(bf16)
    wcm = jnp.stack([_wT(w, C) for w in (w1a, w1b, w2a, w2b)]).astype(bf16)
    wrm = _wT(wr, GR).astype(bf16)
    bm = jnp.stack([b.reshape(C, 1).astype(f32)
                    for b in (b0, b1a, b1b, b2a, b2b)])
    brm = jnp.pad(br, (0, GR - CR)).reshape(GR, 1).astype(f32)

    def body(ct_ref, rs_ref, w0_ref, wc_ref, wr_ref, bm_ref, br_ref,
             fout_ref, rout_ref, x_ref, feat_ref, rec_ref, act_ref, pb_ref):

        def lrelu(v):
            return jnp.where(v >= 0, v, _SLOPE * v)

        def interior(q0):
            l = q0 + lax.broadcasted_iota(jnp.int32, (1, _TN), 1)
            h = l >> 7
            w_ = l & 127
            return (h >= 1) & (h <= H) & (w_ >= 1) & (w_ <= W)

        def rolled_sum(p, gs):
            p0, p1, p2 = p[0:gs, :], p[gs:2 * gs, :], p[2 * gs:3 * gs, :]
            r0 = jnp.concatenate([p0[:, _TN - 1:], p0[:, :_TN - 1]], axis=1)
            r2 = jnp.concatenate([p2[:, 1:], p2[:, :1]], axis=1)
            return r0 + p1 + r2

        def tap_rhs(src_ref, cin, q0):
            v = src_ref[0:cin, pl.ds(_GL - _SL + q0, _TN + 2 * _SL)]
            return jnp.concatenate(
                [v[:, ky * _SL:ky * _SL + _TN] for ky in range(3)], axis=0)

        def conv_pass(src_ref, cin, w, bias, store, gs=C):
            def tile(q0):
                pv = jnp.dot(w, tap_rhs(src_ref, cin, q0),
                             preferred_element_type=f32)
                store(q0, rolled_sum(pv, gs) + bias)

            for j in range(NT):
                tile(j * _TN)

        for ref in (feat_ref, act_ref, pb_ref):
            ref[:, 0:_GL] = jnp.zeros((C, _GL), bf16)
            ref[:, _GL + NPAD:L] = jnp.zeros((C, L - _GL - NPAD), bf16)

        x_ref[:, 0:_GL + _SL] = jnp.zeros((Cin0, _GL + _SL), bf16)
        x_ref[:, _GL + _SL + H * _SL:L] = (
            jnp.zeros((Cin0, L - _GL - _SL - H * _SL), bf16))
        RB = 4 * W
        DB = 4 * _SL

        def pad4(i, carry):
            so = pl.multiple_of(i * RB, 128)
            do = pl.multiple_of(i * DB, 128)
            vc = ct_ref[:, pl.ds(so, RB)].astype(bf16)
            vr = rs_ref[:, pl.ds(so, RB)].astype(bf16)

            def blk(v):
                return jnp.concatenate(
                    [jnp.pad(v[:, j * W:(j + 1) * W], ((0, 0), (1, _SL - W - 1)))
                     for j in range(4)], axis=1)

            x_ref[0:Cc, pl.ds(_GL + _SL + do, DB)] = blk(vc)
            x_ref[Cc:Cin0, pl.ds(_GL + _SL + do, DB)] = blk(vr)
            return carry

        def pad16(i, carry):
            for j in range(4):
                pad4(i * 4 + j, 0)
            return carry
        lax.fori_loop(0, H // 16, pad16, 0)
        for j in range(4 * (H // 16), H // 4):
            pad4(j, 0)

        def st_head(q0, y):
            y = jnp.where(interior(q0), y, 0.0)
            feat_ref[:, pl.ds(_GL + q0, _TN)] = y.astype(bf16)
            act_ref[:, pl.ds(_GL + q0, _TN)] = lrelu(y).astype(bf16)

        def st_mid(q0, y):
            y = jnp.where(interior(q0), lrelu(y), 0.0)
            pb_ref[:, pl.ds(_GL + q0, _TN)] = y.astype(bf16)

        def st_res(q0, y):
            y = (jnp.where(interior(q0), y, 0.0)
                 + feat_ref[:, pl.ds(_GL + q0, _TN)].astype(f32))
            feat_ref[:, pl.ds(_GL + q0, _TN)] = y.astype(bf16)
            act_ref[:, pl.ds(_GL + q0, _TN)] = lrelu(y).astype(bf16)

        def st_rec(q0, y):
            rec_ref[0:CR, pl.ds(_GL + q0, _TN)] = y[0:CR, :].astype(bf16)

        def out4(i, carry):
            do = pl.multiple_of(i * DB, 128)
            so = pl.multiple_of(i * RB, 128)
            vf = feat_ref[:, pl.ds(_GL + _SL + do, DB)]
            vr = rec_ref[0:CR, pl.ds(_GL + _SL + do, DB)]

            def blk(v):
                return jnp.concatenate(
                    [v[:, j * _SL + 1:j * _SL + 1 + W] for j in range(4)],
                    axis=1).astype(f32)

            fout_ref[:, pl.ds(so, RB)] = blk(vf)
            rout_ref[0:CR, pl.ds(so, RB)] = blk(vr)
            return carry

        def out16(i, carry):
            for j in range(4):
                out4(i * 4 + j, 0)
            return carry

        conv_pass(x_ref, Cin0, w0_ref[...], bm_ref[0], st_head)
        conv_pass(act_ref, C, wc_ref[0], bm_ref[1], st_mid)
        conv_pass(pb_ref, C, wc_ref[1], bm_ref[2], st_res)
        conv_pass(act_ref, C, wc_ref[2], bm_ref[3], st_mid)
        conv_pass(pb_ref, C, wc_ref[3], bm_ref[4], st_res)
        conv_pass(feat_ref, C, wr_ref[...], br_ref[...], st_rec, gs=GR)
        lax.fori_loop(0, H // 16, out16, 0)
        for j in range(4 * (H // 16), H // 4):
            out4(j, 0)

    feat, rec = pl.pallas_call(
        body,
        out_shape=(jax.ShapeDtypeStruct((B, C, H * W), f32),
                   jax.ShapeDtypeStruct((B, CR, H * W), f32)),
        grid=(B,),
        in_specs=[
            pl.BlockSpec((None, Cc, H * W), lambda b: (b, 0, 0)),
            pl.BlockSpec((None, Cr, H * W), lambda b: (b, 0, 0)),
            pl.BlockSpec((MW, KW0), lambda b: (0, 0)),
            pl.BlockSpec((4, MW, KWC), lambda b: (0, 0, 0)),
            pl.BlockSpec((3 * GR, KWC), lambda b: (0, 0)),
            pl.BlockSpec((5, C, 1), lambda b: (0, 0, 0)),
            pl.BlockSpec((GR, 1), lambda b: (0, 0)),
        ],
        out_specs=(pl.BlockSpec((None, C, H * W), lambda b: (b, 0, 0)),
                   pl.BlockSpec((None, CR, H * W), lambda b: (b, 0, 0))),
        scratch_shapes=[pltpu.VMEM((Cin0, L), bf16),
                        pltpu.VMEM((C, L), bf16),
                        pltpu.VMEM((8, L), bf16),
                        pltpu.VMEM((C, L), bf16),
                        pltpu.VMEM((C, L), bf16)],
        compiler_params=pltpu.CompilerParams(
            dimension_semantics=("parallel",),
            vmem_limit_bytes=100 << 20),
    )(ctf, ref_, w0m, wcm, wrm, bm, brm)

    return (feat.reshape(B, C, H, W).astype(ctx.dtype),
            rec.reshape(B, CR, H, W).astype(ctx.dtype))


# speedup vs baseline: 7.4702x; 1.0781x over previous
import jax
import jax.numpy as jnp
from jax import lax
from jax.experimental import pallas as pl
from jax.experimental.pallas import tpu as pltpu

_SLOPE = 0.01
_SL = 128
_TN = 896
_GL = 128


def kernel(w0, b0, w1a, b1a, w1b, b1b, w2a, b2a, w2b, b2b, wr, br, ctx, res):
    B, Cc, H, W = ctx.shape
    Cr = res.shape[1]
    Cin0 = Cc + Cr
    C = w0.shape[-1]
    CR = wr.shape[-1]
    Hp = H + 2
    Lp = Hp * _SL
    NT = 2 * (-(-Lp // (2 * _TN)))
    NPAD = NT * _TN
    L = _GL + NPAD + _GL
    KW0 = 3 * Cin0
    KWC = 3 * C
    MW = 3 * C
    f32 = jnp.float32
    bf16 = jnp.bfloat16

    ctf = ctx.reshape(B, Cc, H * W)
    ref_ = res.reshape(B, Cr, H * W)

    def _wT(w, gs):
        cin, cout = w.shape[2], w.shape[3]
        wp = jnp.pad(w, ((0, 0), (0, 0), (0, 0), (0, gs - cout)))
        return jnp.transpose(wp, (1, 3, 0, 2)).reshape(3 * gs, 3 * cin)

    GR = 8
    w0m = _wT(w0, C).astype(bf16)
    wcm = jnp.stack([_wT(w, C) for w in (w1a, w1b, w2a, w2b)]).astype(bf16)
    wrm = _wT(wr, GR).astype(bf16)
    bm = jnp.stack([b.reshape(C, 1).astype(f32)
                    for b in (b0, b1a, b1b, b2a, b2b)])
    brm = jnp.pad(br, (0, GR - CR)).reshape(GR, 1).astype(f32)

    def body(ct_ref, rs_ref, w0_ref, wc_ref, wr_ref, bm_ref, br_ref,
             fout_ref, rout_ref, x_ref, feat_ref, rec_ref, act_ref, pb_ref):

        def lrelu(v):
            return jnp.where(v >= 0, v, _SLOPE * v)

        def interior(q0):
            l = q0 + lax.broadcasted_iota(jnp.int32, (1, _TN), 1)
            h = l >> 7
            w_ = l & 127
            return (h >= 1) & (h <= H) & (w_ >= 1) & (w_ <= W)

        def rolled_sum(p, gs):
            p0, p1, p2 = p[0:gs, :], p[gs:2 * gs, :], p[2 * gs:3 * gs, :]
            r0 = jnp.concatenate([p0[:, _TN - 1:], p0[:, :_TN - 1]], axis=1)
            r2 = jnp.concatenate([p2[:, 1:], p2[:, :1]], axis=1)
            return r0 + p1 + r2

        def tap_rhs(src_ref, cin, q0):
            v = src_ref[0:cin, pl.ds(_GL - _SL + q0, _TN + 2 * _SL)]
            return jnp.concatenate(
                [v[:, ky * _SL:ky * _SL + _TN] for ky in range(3)], axis=0)

        def conv_pass(src_ref, cin, w, bias, store, gs=C):
            def tile(q0):
                pv = jnp.dot(w, tap_rhs(src_ref, cin, q0),
                             preferred_element_type=f32)
                store(q0, rolled_sum(pv, gs) + bias)

            for j in range(NT):
                tile(j * _TN)

        for ref in (feat_ref, act_ref, pb_ref):
            ref[:, 0:_GL] = jnp.zeros((C, _GL), bf16)
            ref[:, _GL + NPAD:L] = jnp.zeros((C, L - _GL - NPAD), bf16)

        x_ref[:, 0:_GL + _SL] = jnp.zeros((Cin0, _GL + _SL), bf16)
        x_ref[:, _GL + _SL + H * _SL:L] = (
            jnp.zeros((Cin0, L - _GL - _SL - H * _SL), bf16))
        RB = 4 * W
        DB = 4 * _SL

        def pad4(i, carry):
            so = pl.multiple_of(i * RB, 128)
            do = pl.multiple_of(i * DB, 128)
            vc = ct_ref[:, pl.ds(so, RB)].astype(bf16)
            vr = rs_ref[:, pl.ds(so, RB)].astype(bf16)

            def blk(v):
                return jnp.concatenate(
                    [jnp.pad(v[:, j * W:(j + 1) * W], ((0, 0), (1, _SL - W - 1)))
                     for j in range(4)], axis=1)

            x_ref[0:Cc, pl.ds(_GL + _SL + do, DB)] = blk(vc)
            x_ref[Cc:Cin0, pl.ds(_GL + _SL + do, DB)] = blk(vr)
            return carry

        for j in range(H // 4):
            pad4(j, 0)

        def st_head(q0, y):
            y = jnp.where(interior(q0), y, 0.0)
            feat_ref[:, pl.ds(_GL + q0, _TN)] = y.astype(bf16)
            act_ref[:, pl.ds(_GL + q0, _TN)] = lrelu(y).astype(bf16)

        def st_mid(q0, y):
            y = jnp.where(interior(q0), lrelu(y), 0.0)
            pb_ref[:, pl.ds(_GL + q0, _TN)] = y.astype(bf16)

        def st_res(q0, y):
            y = (jnp.where(interior(q0), y, 0.0)
                 + feat_ref[:, pl.ds(_GL + q0, _TN)].astype(f32))
            feat_ref[:, pl.ds(_GL + q0, _TN)] = y.astype(bf16)
            act_ref[:, pl.ds(_GL + q0, _TN)] = lrelu(y).astype(bf16)

        def st_rec(q0, y):
            rec_ref[0:CR, pl.ds(_GL + q0, _TN)] = y[0:CR, :].astype(bf16)

        def out4(i, carry):
            do = pl.multiple_of(i * DB, 128)
            so = pl.multiple_of(i * RB, 128)
            vf = feat_ref[:, pl.ds(_GL + _SL + do, DB)]
            vr = rec_ref[0:CR, pl.ds(_GL + _SL + do, DB)]

            def blk(v):
                return jnp.concatenate(
                    [v[:, j * _SL + 1:j * _SL + 1 + W] for j in range(4)],
                    axis=1).astype(f32)

            fout_ref[:, pl.ds(so, RB)] = blk(vf)
            rout_ref[0:CR, pl.ds(so, RB)] = blk(vr)
            return carry


        conv_pass(x_ref, Cin0, w0_ref[...], bm_ref[0], st_head)
        conv_pass(act_ref, C, wc_ref[0], bm_ref[1], st_mid)
        conv_pass(pb_ref, C, wc_ref[1], bm_ref[2], st_res)
        conv_pass(act_ref, C, wc_ref[2], bm_ref[3], st_mid)
        conv_pass(pb_ref, C, wc_ref[3], bm_ref[4], st_res)
        conv_pass(feat_ref, C, wr_ref[...], br_ref[...], st_rec, gs=GR)
        for j in range(H // 4):
            out4(j, 0)

    feat, rec = pl.pallas_call(
        body,
        out_shape=(jax.ShapeDtypeStruct((B, C, H * W), f32),
                   jax.ShapeDtypeStruct((B, CR, H * W), f32)),
        grid=(B,),
        in_specs=[
            pl.BlockSpec((None, Cc, H * W), lambda b: (b, 0, 0)),
            pl.BlockSpec((None, Cr, H * W), lambda b: (b, 0, 0)),
            pl.BlockSpec((MW, KW0), lambda b: (0, 0)),
            pl.BlockSpec((4, MW, KWC), lambda b: (0, 0, 0)),
            pl.BlockSpec((3 * GR, KWC), lambda b: (0, 0)),
            pl.BlockSpec((5, C, 1), lambda b: (0, 0, 0)),
            pl.BlockSpec((GR, 1), lambda b: (0, 0)),
        ],
        out_specs=(pl.BlockSpec((None, C, H * W), lambda b: (b, 0, 0)),
                   pl.BlockSpec((None, CR, H * W), lambda b: (b, 0, 0))),
        scratch_shapes=[pltpu.VMEM((Cin0, L), bf16),
                        pltpu.VMEM((C, L), bf16),
                        pltpu.VMEM((8, L), bf16),
                        pltpu.VMEM((C, L), bf16),
                        pltpu.VMEM((C, L), bf16)],
        compiler_params=pltpu.CompilerParams(
            dimension_semantics=("parallel",),
            vmem_limit_bytes=100 << 20),
    )(ctf, ref_, w0m, wcm, wrm, bm, brm)

    return (feat.reshape(B, C, H, W).astype(ctx.dtype),
            rec.reshape(B, CR, H, W).astype(ctx.dtype))
